# baseline (speedup 1.0000x reference)
import functools

import jax
import jax.numpy as jnp
from jax import lax
from jax.experimental import pallas as pl
from jax.experimental.pallas import tpu as pltpu
from jax.experimental.pallas import tpu_sc as plsc

N, K, D, F, O = 10000, 16, 256, 16, 256
H = D // 2

NW = 32
NP = 10240
NPS = 8
IDXROW = NPS * K
S_FAST = 80
S_SLOW = 0
LANES = 16
IDXTOT = NP * K // IDXROW
IDXPAD = IDXTOT + S_FAST - S_SLOW

_HI = -65536


def _sc_gather_sum(node_hbm, hood_hbm, out_hbm, idx_v, rows_a, rows_b, rows_c,
                   rows_d, acc_v, sem_a, sem_b, sem_c, sem_d):
    c = lax.axis_index("c")
    s = lax.axis_index("s")
    ring = ((rows_a, sem_a), (rows_b, sem_b), (rows_c, sem_c), (rows_d, sem_d))

    def worker(steps, base_idx_row):
        base_node = base_idx_row * NPS
        pltpu.sync_copy(hood_hbm.at[pl.ds(base_idx_row, steps)],
                        idx_v.at[pl.ds(0, steps)])

        def fire(g, rows, sem):
            pltpu.make_async_copy(node_hbm.at[idx_v.at[g]], rows, sem).start()

        def drain(rows, sem):
            pltpu.make_async_copy(node_hbm.at[idx_v.at[0]], rows, sem).wait()

        def reduce_rows(g, rows):
            def node_body(j, carry2):
                r0 = j * K
                for cg in range(H // LANES):
                    sl = pl.ds(cg * LANES, LANES)
                    w = rows[r0, sl]
                    acc_lo = lax.bitcast_convert_type(w << 16, jnp.float32)
                    acc_hi = lax.bitcast_convert_type(w & _HI, jnp.float32)
                    for k in range(1, K):
                        w = rows[r0 + k, sl]
                        acc_lo = acc_lo + lax.bitcast_convert_type(
                            w << 16, jnp.float32)
                        acc_hi = acc_hi + lax.bitcast_convert_type(
                            w & _HI, jnp.float32)
                    lo_w = lax.shift_right_logical(
                        lax.bitcast_convert_type(acc_lo, jnp.int32), 16)
                    hi_w = lax.bitcast_convert_type(acc_hi, jnp.int32) & _HI
                    acc_v[j, sl] = hi_w | lo_w
                return carry2

            lax.fori_loop(0, NPS, node_body, 0)
            pltpu.sync_copy(acc_v, out_hbm.at[pl.ds(base_node + g * NPS, NPS)])

        for b, (rows, sem) in enumerate(ring):
            fire(b, rows, sem)

        def round4(p, carry):
            g0 = 4 * p
            for b, (rows, sem) in enumerate(ring):
                drain(rows, sem)
                reduce_rows(g0 + b, rows)
                fire(g0 + b + 4, rows, sem)
            return carry

        lax.fori_loop(0, steps // 4 - 1, round4, 0)
        for b, (rows, sem) in enumerate(ring):
            drain(rows, sem)
            reduce_rows(steps - 4 + b, rows)

    pl.when(c == 0)(lambda: worker(S_FAST, s * S_FAST))
    if S_SLOW:
        pl.when(c == 1)(lambda: worker(S_SLOW, 16 * S_FAST + s * S_SLOW))


_gather_sum = functools.partial(
    pl.kernel,
    mesh=plsc.VectorSubcoreMesh(core_axis_name="c", subcore_axis_name="s"),
    out_type=jax.ShapeDtypeStruct((NP, H), jnp.int32),
    scratch_types=[
        pltpu.VMEM((S_FAST, IDXROW), jnp.int32),
        pltpu.VMEM((IDXROW, H), jnp.int32),
        pltpu.VMEM((IDXROW, H), jnp.int32),
        pltpu.VMEM((IDXROW, H), jnp.int32),
        pltpu.VMEM((IDXROW, H), jnp.int32),
        pltpu.VMEM((NPS, H), jnp.int32),
        pltpu.SemaphoreType.DMA,
        pltpu.SemaphoreType.DMA,
        pltpu.SemaphoreType.DMA,
        pltpu.SemaphoreType.DMA,
    ],
)(_sc_gather_sum)


BT = 400
_RND = 0x8000


def _pack_body(node_ref, out_ref):
    x = node_ref[...]
    lo = lax.bitcast_convert_type(x[:, :H], jnp.int32) + _RND
    hi = lax.bitcast_convert_type(x[:, H:], jnp.int32) + _RND
    out_ref[...] = (hi & _HI) | lax.shift_right_logical(lo, 16)


def _tc_body(node_ref, sw_ref, edge_ref, hood_ref, wc_ref, wn_ref, we_ref,
             out_ref):
    h = hood_ref[...]
    cnt = jnp.sum((h != -1).astype(jnp.float32), axis=1, keepdims=True)
    denom = jnp.maximum(cnt, 1.0)
    es = jnp.sum(edge_ref[...], axis=1)
    w = sw_ref[...]
    s_lo = lax.bitcast_convert_type(w << 16, jnp.float32)
    s_hi = lax.bitcast_convert_type(w & _HI, jnp.float32)
    t1 = jnp.dot(node_ref[...], wc_ref[...],
                 preferred_element_type=jnp.float32)
    t2 = (jnp.dot(s_lo, wn_ref[:H, :], preferred_element_type=jnp.float32)
          + jnp.dot(s_hi, wn_ref[H:, :], preferred_element_type=jnp.float32))
    t3 = jnp.dot(es, we_ref[...], preferred_element_type=jnp.float32)
    out_ref[...] = t1 + (t2 + t3) / denom


def kernel(node, edge, hood, w_node_c, w_node_n, w_edge):
    hood2 = hood.reshape(N, K)
    hood_pad = jnp.concatenate(
        [hood2.reshape(N * K // IDXROW, IDXROW),
         jnp.zeros((IDXPAD - N * K // IDXROW, IDXROW), jnp.int32)], axis=0)

    node_w = pl.pallas_call(
        _pack_body,
        grid=(N // BT,),
        in_specs=[pl.BlockSpec((BT, D), lambda i: (i, 0))],
        out_specs=pl.BlockSpec((BT, H), lambda i: (i, 0)),
        out_shape=jax.ShapeDtypeStruct((N, H), jnp.int32),
    )(node)

    s_w = _gather_sum(node_w, hood_pad)

    out = pl.pallas_call(
        _tc_body,
        grid=(N // BT,),
        in_specs=[
            pl.BlockSpec((BT, D), lambda i: (i, 0)),
            pl.BlockSpec((BT, H), lambda i: (i, 0)),
            pl.BlockSpec((BT, K, F), lambda i: (i, 0, 0)),
            pl.BlockSpec((BT, K), lambda i: (i, 0)),
            pl.BlockSpec((D, O), lambda i: (0, 0)),
            pl.BlockSpec((D, O), lambda i: (0, 0)),
            pl.BlockSpec((F, O), lambda i: (0, 0)),
        ],
        out_specs=pl.BlockSpec((BT, O), lambda i: (i, 0)),
        out_shape=jax.ShapeDtypeStruct((N, O), jnp.float32),
    )(node, s_w, edge, hood2, w_node_c, w_node_n, w_edge)
    return out


# speedup vs baseline: 1.2512x; 1.2512x over previous
import functools

import jax
import jax.numpy as jnp
from jax import lax
from jax.experimental import pallas as pl
from jax.experimental.pallas import tpu as pltpu
from jax.experimental.pallas import tpu_sc as plsc

N, K, D, F, O = 10000, 16, 256, 16, 256
H = D // 2

NW = 32
NP = 10240
NPS = 8
IDXROW = NPS * K
S_FAST = 72
S_SLOW = 8
LANES = 16
IDXTOT = NP * K // IDXROW
IDXPAD = IDXTOT + S_FAST - S_SLOW

_HI = -65536


def _sc_gather_sum(node_hbm, hood_hbm, out_hbm, idx_v, rows_a, rows_b, rows_c,
                   rows_d, acc_v, sem_a, sem_b, sem_c, sem_d):
    c = lax.axis_index("c")
    s = lax.axis_index("s")
    ring = ((rows_a, sem_a), (rows_b, sem_b), (rows_c, sem_c), (rows_d, sem_d))

    def worker(steps, base_idx_row):
        base_node = base_idx_row * NPS
        pltpu.sync_copy(hood_hbm.at[pl.ds(base_idx_row, S_FAST)], idx_v)

        def fire(g, rows, sem):
            pltpu.make_async_copy(node_hbm.at[idx_v.at[g]], rows, sem).start()

        def drain(rows, sem):
            pltpu.make_async_copy(node_hbm.at[idx_v.at[0]], rows, sem).wait()

        def reduce_rows(g, rows):
            def node_body(j, carry2):
                r0 = j * K
                for cg in range(H // LANES):
                    sl = pl.ds(cg * LANES, LANES)
                    w = rows[r0, sl]
                    acc_lo = lax.bitcast_convert_type(w << 16, jnp.float32)
                    acc_hi = lax.bitcast_convert_type(w & _HI, jnp.float32)
                    for k in range(1, K):
                        w = rows[r0 + k, sl]
                        acc_lo = acc_lo + lax.bitcast_convert_type(
                            w << 16, jnp.float32)
                        acc_hi = acc_hi + lax.bitcast_convert_type(
                            w & _HI, jnp.float32)
                    lo_w = lax.shift_right_logical(
                        lax.bitcast_convert_type(acc_lo, jnp.int32), 16)
                    hi_w = lax.bitcast_convert_type(acc_hi, jnp.int32) & _HI
                    acc_v[j, sl] = hi_w | lo_w
                return carry2

            lax.fori_loop(0, NPS, node_body, 0)
            pltpu.sync_copy(acc_v, out_hbm.at[pl.ds(base_node + g * NPS, NPS)])

        for b, (rows, sem) in enumerate(ring):
            fire(b, rows, sem)

        def round4(p, carry):
            g0 = 4 * p
            for b, (rows, sem) in enumerate(ring):
                drain(rows, sem)
                reduce_rows(g0 + b, rows)
                fire(g0 + b + 4, rows, sem)
            return carry

        lax.fori_loop(0, steps // 4 - 1, round4, 0)
        for b, (rows, sem) in enumerate(ring):
            drain(rows, sem)
            reduce_rows(steps - 4 + b, rows)

    fast = c == 0
    steps_w = jnp.where(fast, S_FAST, S_SLOW)
    base_w = jnp.where(fast, s * S_FAST, 16 * S_FAST + s * S_SLOW)
    if S_SLOW:
        worker(steps_w, base_w)
    else:
        pl.when(fast)(lambda: worker(S_FAST, s * S_FAST))


_gather_sum = functools.partial(
    pl.kernel,
    mesh=plsc.VectorSubcoreMesh(core_axis_name="c", subcore_axis_name="s"),
    out_type=jax.ShapeDtypeStruct((NP, H), jnp.int32),
    scratch_types=[
        pltpu.VMEM((S_FAST, IDXROW), jnp.int32),
        pltpu.VMEM((IDXROW, H), jnp.int32),
        pltpu.VMEM((IDXROW, H), jnp.int32),
        pltpu.VMEM((IDXROW, H), jnp.int32),
        pltpu.VMEM((IDXROW, H), jnp.int32),
        pltpu.VMEM((NPS, H), jnp.int32),
        pltpu.SemaphoreType.DMA,
        pltpu.SemaphoreType.DMA,
        pltpu.SemaphoreType.DMA,
        pltpu.SemaphoreType.DMA,
    ],
)(_sc_gather_sum)


BT = 400
_RND = 0x8000


def _pack_body(node_ref, out_ref):
    x = node_ref[...]
    lo = lax.bitcast_convert_type(x[:, :H], jnp.int32) + _RND
    hi = lax.bitcast_convert_type(x[:, H:], jnp.int32) + _RND
    out_ref[...] = (hi & _HI) | lax.shift_right_logical(lo, 16)


def _tc_body(node_ref, sw_ref, edge_ref, hood_ref, wc_ref, wn_ref, we_ref,
             out_ref):
    h = hood_ref[...]
    cnt = jnp.sum((h != -1).astype(jnp.float32), axis=1, keepdims=True)
    denom = jnp.maximum(cnt, 1.0)
    es = jnp.sum(edge_ref[...], axis=1)
    w = sw_ref[...]
    s_lo = lax.bitcast_convert_type(w << 16, jnp.float32)
    s_hi = lax.bitcast_convert_type(w & _HI, jnp.float32)
    t1 = jnp.dot(node_ref[...], wc_ref[...],
                 preferred_element_type=jnp.float32)
    t2 = (jnp.dot(s_lo, wn_ref[:H, :], preferred_element_type=jnp.float32)
          + jnp.dot(s_hi, wn_ref[H:, :], preferred_element_type=jnp.float32))
    t3 = jnp.dot(es, we_ref[...], preferred_element_type=jnp.float32)
    out_ref[...] = t1 + (t2 + t3) / denom


def kernel(node, edge, hood, w_node_c, w_node_n, w_edge):
    hood2 = hood.reshape(N, K)
    hood_pad = jnp.concatenate(
        [hood2.reshape(N * K // IDXROW, IDXROW),
         jnp.zeros((IDXPAD - N * K // IDXROW, IDXROW), jnp.int32)], axis=0)

    node_w = pl.pallas_call(
        _pack_body,
        grid=(N // BT,),
        in_specs=[pl.BlockSpec((BT, D), lambda i: (i, 0))],
        out_specs=pl.BlockSpec((BT, H), lambda i: (i, 0)),
        out_shape=jax.ShapeDtypeStruct((N, H), jnp.int32),
    )(node)

    s_w = _gather_sum(node_w, hood_pad)

    out = pl.pallas_call(
        _tc_body,
        grid=(N // BT,),
        in_specs=[
            pl.BlockSpec((BT, D), lambda i: (i, 0)),
            pl.BlockSpec((BT, H), lambda i: (i, 0)),
            pl.BlockSpec((BT, K, F), lambda i: (i, 0, 0)),
            pl.BlockSpec((BT, K), lambda i: (i, 0)),
            pl.BlockSpec((D, O), lambda i: (0, 0)),
            pl.BlockSpec((D, O), lambda i: (0, 0)),
            pl.BlockSpec((F, O), lambda i: (0, 0)),
        ],
        out_specs=pl.BlockSpec((BT, O), lambda i: (i, 0)),
        out_shape=jax.ShapeDtypeStruct((N, O), jnp.float32),
    )(node, s_w, edge, hood2, w_node_c, w_node_n, w_edge)
    return out


# speedup vs baseline: 1.2533x; 1.0017x over previous
import functools

import jax
import jax.numpy as jnp
from jax import lax
from jax.experimental import pallas as pl
from jax.experimental.pallas import tpu as pltpu
from jax.experimental.pallas import tpu_sc as plsc

N, K, D, F, O = 10000, 16, 256, 16, 256
H = D // 2

NW = 32
NP = 10240
NPS = 8
IDXROW = NPS * K
S_FAST = 72
S_SLOW = 8
LANES = 16
IDXTOT = NP * K // IDXROW
IDXPAD = IDXTOT + S_FAST - S_SLOW

_HI = -65536


def _sc_gather_sum(node_hbm, hood_hbm, out_hbm, idx_v, rows_a, rows_b, rows_c,
                   rows_d, acc_v, sem_a, sem_b, sem_c, sem_d):
    c = lax.axis_index("c")
    s = lax.axis_index("s")
    ring = ((rows_a, sem_a), (rows_b, sem_b), (rows_c, sem_c), (rows_d, sem_d))

    def worker(steps, base_idx_row):
        base_node = base_idx_row * NPS
        pltpu.sync_copy(hood_hbm.at[pl.ds(base_idx_row, S_FAST)], idx_v)

        def fire(g, rows, sem):
            pltpu.make_async_copy(node_hbm.at[idx_v.at[g]], rows, sem).start()

        def drain(rows, sem):
            pltpu.make_async_copy(node_hbm.at[idx_v.at[0]], rows, sem).wait()

        def reduce_rows(g, rows):
            def node_body(j, carry2):
                r0 = j * K
                for cg in range(H // LANES):
                    sl = pl.ds(cg * LANES, LANES)
                    w = rows[r0, sl]
                    acc_lo = lax.bitcast_convert_type(w << 16, jnp.float32)
                    acc_hi = lax.bitcast_convert_type(w & _HI, jnp.float32)
                    for k in range(1, K):
                        w = rows[r0 + k, sl]
                        acc_lo = acc_lo + lax.bitcast_convert_type(
                            w << 16, jnp.float32)
                        acc_hi = acc_hi + lax.bitcast_convert_type(
                            w & _HI, jnp.float32)
                    lo_w = lax.shift_right_logical(
                        lax.bitcast_convert_type(acc_lo, jnp.int32), 16)
                    hi_w = lax.bitcast_convert_type(acc_hi, jnp.int32) & _HI
                    acc_v[j, sl] = hi_w | lo_w
                return carry2

            lax.fori_loop(0, NPS, node_body, 0)
            pltpu.sync_copy(acc_v, out_hbm.at[pl.ds(base_node + g * NPS, NPS)])

        for b, (rows, sem) in enumerate(ring):
            fire(b, rows, sem)

        def round4(p, carry):
            g0 = 4 * p
            for b, (rows, sem) in enumerate(ring):
                drain(rows, sem)
                reduce_rows(g0 + b, rows)
                fire(g0 + b + 4, rows, sem)
            return carry

        lax.fori_loop(0, steps // 4 - 1, round4, 0)
        for b, (rows, sem) in enumerate(ring):
            drain(rows, sem)
            reduce_rows(steps - 4 + b, rows)

    fast = c == 0
    steps_w = jnp.where(fast, S_SLOW, S_FAST)
    base_w = jnp.where(fast, 16 * S_FAST + s * S_SLOW, s * S_FAST)
    if S_SLOW:
        worker(steps_w, base_w)
    else:
        pl.when(fast)(lambda: worker(S_FAST, s * S_FAST))


_gather_sum = functools.partial(
    pl.kernel,
    mesh=plsc.VectorSubcoreMesh(core_axis_name="c", subcore_axis_name="s"),
    out_type=jax.ShapeDtypeStruct((NP, H), jnp.int32),
    scratch_types=[
        pltpu.VMEM((S_FAST, IDXROW), jnp.int32),
        pltpu.VMEM((IDXROW, H), jnp.int32),
        pltpu.VMEM((IDXROW, H), jnp.int32),
        pltpu.VMEM((IDXROW, H), jnp.int32),
        pltpu.VMEM((IDXROW, H), jnp.int32),
        pltpu.VMEM((NPS, H), jnp.int32),
        pltpu.SemaphoreType.DMA,
        pltpu.SemaphoreType.DMA,
        pltpu.SemaphoreType.DMA,
        pltpu.SemaphoreType.DMA,
    ],
)(_sc_gather_sum)


BT = 400
_RND = 0x8000


def _pack_body(node_ref, out_ref):
    x = node_ref[...]
    lo = lax.bitcast_convert_type(x[:, :H], jnp.int32) + _RND
    hi = lax.bitcast_convert_type(x[:, H:], jnp.int32) + _RND
    out_ref[...] = (hi & _HI) | lax.shift_right_logical(lo, 16)


def _tc_body(node_ref, sw_ref, edge_ref, hood_ref, wc_ref, wn_ref, we_ref,
             out_ref):
    h = hood_ref[...]
    cnt = jnp.sum((h != -1).astype(jnp.float32), axis=1, keepdims=True)
    denom = jnp.maximum(cnt, 1.0)
    es = jnp.sum(edge_ref[...], axis=1)
    w = sw_ref[...]
    s_lo = lax.bitcast_convert_type(w << 16, jnp.float32)
    s_hi = lax.bitcast_convert_type(w & _HI, jnp.float32)
    t1 = jnp.dot(node_ref[...], wc_ref[...],
                 preferred_element_type=jnp.float32)
    t2 = (jnp.dot(s_lo, wn_ref[:H, :], preferred_element_type=jnp.float32)
          + jnp.dot(s_hi, wn_ref[H:, :], preferred_element_type=jnp.float32))
    t3 = jnp.dot(es, we_ref[...], preferred_element_type=jnp.float32)
    out_ref[...] = t1 + (t2 + t3) / denom


def kernel(node, edge, hood, w_node_c, w_node_n, w_edge):
    hood2 = hood.reshape(N, K)
    hood_pad = jnp.concatenate(
        [hood2.reshape(N * K // IDXROW, IDXROW),
         jnp.zeros((IDXPAD - N * K // IDXROW, IDXROW), jnp.int32)], axis=0)

    node_w = pl.pallas_call(
        _pack_body,
        grid=(N // BT,),
        in_specs=[pl.BlockSpec((BT, D), lambda i: (i, 0))],
        out_specs=pl.BlockSpec((BT, H), lambda i: (i, 0)),
        out_shape=jax.ShapeDtypeStruct((N, H), jnp.int32),
    )(node)

    s_w = _gather_sum(node_w, hood_pad)

    out = pl.pallas_call(
        _tc_body,
        grid=(N // BT,),
        in_specs=[
            pl.BlockSpec((BT, D), lambda i: (i, 0)),
            pl.BlockSpec((BT, H), lambda i: (i, 0)),
            pl.BlockSpec((BT, K, F), lambda i: (i, 0, 0)),
            pl.BlockSpec((BT, K), lambda i: (i, 0)),
            pl.BlockSpec((D, O), lambda i: (0, 0)),
            pl.BlockSpec((D, O), lambda i: (0, 0)),
            pl.BlockSpec((F, O), lambda i: (0, 0)),
        ],
        out_specs=pl.BlockSpec((BT, O), lambda i: (i, 0)),
        out_shape=jax.ShapeDtypeStruct((N, O), jnp.float32),
    )(node, s_w, edge, hood2, w_node_c, w_node_n, w_edge)
    return out


# speedup vs baseline: 1.3024x; 1.0392x over previous
import functools

import jax
import jax.numpy as jnp
from jax import lax
from jax.experimental import pallas as pl
from jax.experimental.pallas import tpu as pltpu
from jax.experimental.pallas import tpu_sc as plsc

N, K, D, F, O = 10000, 16, 256, 16, 256
H = D // 2

NW = 32
NP = 10240
NPS = 8
IDXROW = NPS * K
S_FAST = 72
S_SLOW = 8
LANES = 16
IDXTOT = NP * K // IDXROW
IDXPAD = IDXTOT + S_FAST - S_SLOW

_HI = -65536


def _sc_gather_sum(node_hbm, hood_hbm, out_hbm, idx_v, rows_a, rows_b, rows_c,
                   rows_d, acc_v, sem_a, sem_b, sem_c, sem_d):
    c = lax.axis_index("c")
    s = lax.axis_index("s")
    ring = ((rows_a, sem_a), (rows_b, sem_b), (rows_c, sem_c), (rows_d, sem_d))

    def worker(steps, base_idx_row):
        base_node = base_idx_row * NPS
        pltpu.sync_copy(hood_hbm.at[pl.ds(base_idx_row, S_FAST)], idx_v)

        def fire(g, rows, sem):
            pltpu.make_async_copy(node_hbm.at[idx_v.at[g]], rows, sem).start()

        def drain(rows, sem):
            pltpu.make_async_copy(node_hbm.at[idx_v.at[0]], rows, sem).wait()

        def reduce_rows(g, rows):
            def node_body(j, carry2):
                r0 = j * K
                for cg in range(H // LANES):
                    sl = pl.ds(cg * LANES, LANES)
                    w = rows[r0, sl]
                    acc_lo = lax.bitcast_convert_type(w << 16, jnp.float32)
                    acc_hi = lax.bitcast_convert_type(w & _HI, jnp.float32)
                    for k in range(1, K):
                        w = rows[r0 + k, sl]
                        acc_lo = acc_lo + lax.bitcast_convert_type(
                            w << 16, jnp.float32)
                        acc_hi = acc_hi + lax.bitcast_convert_type(
                            w & _HI, jnp.float32)
                    lo_w = lax.shift_right_logical(
                        lax.bitcast_convert_type(acc_lo, jnp.int32), 16)
                    hi_w = lax.bitcast_convert_type(acc_hi, jnp.int32) & _HI
                    acc_v[j, sl] = hi_w | lo_w
                return carry2

            lax.fori_loop(0, NPS, node_body, 0)
            pltpu.sync_copy(acc_v, out_hbm.at[pl.ds(base_node + g * NPS, NPS)])

        for b, (rows, sem) in enumerate(ring):
            fire(b, rows, sem)

        def round4(p, carry):
            g0 = 4 * p
            for b, (rows, sem) in enumerate(ring):
                drain(rows, sem)
                reduce_rows(g0 + b, rows)
                fire(g0 + b + 4, rows, sem)
            return carry

        lax.fori_loop(0, steps // 4 - 1, round4, 0)
        for b, (rows, sem) in enumerate(ring):
            drain(rows, sem)
            reduce_rows(steps - 4 + b, rows)

    fast = c == 0
    steps_w = jnp.where(fast, S_SLOW, S_FAST)
    base_w = jnp.where(fast, 16 * S_FAST + s * S_SLOW, s * S_FAST)
    if S_SLOW:
        worker(steps_w, base_w)
    else:
        pl.when(fast)(lambda: worker(S_FAST, s * S_FAST))


_gather_sum = functools.partial(
    pl.kernel,
    mesh=plsc.VectorSubcoreMesh(core_axis_name="c", subcore_axis_name="s"),
    out_type=jax.ShapeDtypeStruct((NP, H), jnp.int32),
    scratch_types=[
        pltpu.VMEM((S_FAST, IDXROW), jnp.int32),
        pltpu.VMEM((IDXROW, H), jnp.int32),
        pltpu.VMEM((IDXROW, H), jnp.int32),
        pltpu.VMEM((IDXROW, H), jnp.int32),
        pltpu.VMEM((IDXROW, H), jnp.int32),
        pltpu.VMEM((NPS, H), jnp.int32),
        pltpu.SemaphoreType.DMA,
        pltpu.SemaphoreType.DMA,
        pltpu.SemaphoreType.DMA,
        pltpu.SemaphoreType.DMA,
    ],
)(_sc_gather_sum)


BT = 2000
BP = 1000
_RND = 0x8000


def _pack_body(node_ref, out_ref):
    x = node_ref[...]
    lo = lax.bitcast_convert_type(x[:, :H], jnp.int32) + _RND
    hi = lax.bitcast_convert_type(x[:, H:], jnp.int32) + _RND
    out_ref[...] = (hi & _HI) | lax.shift_right_logical(lo, 16)


def _tc_body(node_ref, sw_ref, edge_ref, hood_ref, wc_ref, wn_ref, we_ref,
             out_ref):
    h = hood_ref[...]
    cnt = jnp.sum((h != -1).astype(jnp.float32), axis=1, keepdims=True)
    denom = jnp.maximum(cnt, 1.0)
    es = jnp.sum(edge_ref[...], axis=1)
    w = sw_ref[...]
    s_lo = lax.bitcast_convert_type(w << 16, jnp.float32)
    s_hi = lax.bitcast_convert_type(w & _HI, jnp.float32)
    t1 = jnp.dot(node_ref[...], wc_ref[...],
                 preferred_element_type=jnp.float32)
    t2 = (jnp.dot(s_lo, wn_ref[:H, :], preferred_element_type=jnp.float32)
          + jnp.dot(s_hi, wn_ref[H:, :], preferred_element_type=jnp.float32))
    t3 = jnp.dot(es, we_ref[...], preferred_element_type=jnp.float32)
    out_ref[...] = t1 + (t2 + t3) / denom


def kernel(node, edge, hood, w_node_c, w_node_n, w_edge):
    hood2 = hood.reshape(N, K)
    hood_pad = jnp.concatenate(
        [hood2.reshape(N * K // IDXROW, IDXROW),
         jnp.zeros((IDXPAD - N * K // IDXROW, IDXROW), jnp.int32)], axis=0)

    node_w = pl.pallas_call(
        _pack_body,
        grid=(N // BP,),
        in_specs=[pl.BlockSpec((BP, D), lambda i: (i, 0))],
        out_specs=pl.BlockSpec((BP, H), lambda i: (i, 0)),
        out_shape=jax.ShapeDtypeStruct((N, H), jnp.int32),
    )(node)

    s_w = _gather_sum(node_w, hood_pad)

    out = pl.pallas_call(
        _tc_body,
        grid=(N // BT,),
        in_specs=[
            pl.BlockSpec((BT, D), lambda i: (i, 0)),
            pl.BlockSpec((BT, H), lambda i: (i, 0)),
            pl.BlockSpec((BT, K, F), lambda i: (i, 0, 0)),
            pl.BlockSpec((BT, K), lambda i: (i, 0)),
            pl.BlockSpec((D, O), lambda i: (0, 0)),
            pl.BlockSpec((D, O), lambda i: (0, 0)),
            pl.BlockSpec((F, O), lambda i: (0, 0)),
        ],
        out_specs=pl.BlockSpec((BT, O), lambda i: (i, 0)),
        out_shape=jax.ShapeDtypeStruct((N, O), jnp.float32),
    )(node, s_w, edge, hood2, w_node_c, w_node_n, w_edge)
    return out
